# Initial kernel scaffold; baseline (speedup 1.0000x reference)
import jax
import jax.numpy as jnp
from jax.experimental import pallas as pl


def kernel(rot, trans, pair_rot, pair_trans, confidences, topology):
    raise NotImplementedError("write your pallas kernel here")


# speedup vs baseline: 31.8872x; 31.8872x over previous
import functools

import jax
import jax.numpy as jnp
from jax import lax
from jax.experimental import pallas as pl
from jax.experimental.pallas import tpu as pltpu
from jax.experimental.pallas import tpu_sc as plsc

B, N, K = 8, 4096, 32
NC, NS = 2, 16
NW = NC * NS
NPW = (B * N) // NW
WPB = N // NPW
NCH = 64
CHUNKS = NPW // NCH
GROUPS = NCH // 16


def _sc_body(rot_h, trans_h, prot_h, ptrans_h, conf_h, topo_h, out_h,
             rot_v, trans_v, topo_v, conf_v, ptrans_v, prot_v, out_v):
    wid = lax.axis_index("s") * NC + lax.axis_index("c")
    b = wid // WPB
    node0 = (wid % WPB) * NPW

    pltpu.sync_copy(rot_h.at[pl.ds(b * (N * 9), N * 9)], rot_v)
    pltpu.sync_copy(trans_h.at[pl.ds(b * (N * 3), N * 3)], trans_v)

    lane = lax.iota(jnp.int32, 16)
    base_topo = lane * K
    base_prot = lane * (K * 9)
    base_ptr = lane * (K * 3)

    def chunk_body(ch, _):
        start = node0 + ch * NCH
        eg = (b * N + start) * K
        pltpu.sync_copy(topo_h.at[pl.ds(eg, NCH * K)], topo_v)
        pltpu.sync_copy(conf_h.at[pl.ds(eg, NCH * K)], conf_v)
        pltpu.sync_copy(ptrans_h.at[pl.ds(eg * 3, NCH * K * 3)], ptrans_v)
        pltpu.sync_copy(prot_h.at[pl.ds(eg * 9, NCH * K * 9)], prot_v)

        def group_body(g, _):
            goff = g * 16
            zero = jnp.zeros((16,), jnp.float32)
            acc_r = [zero] * 9
            acc_t = [zero] * 3
            acc_w = zero
            t_off = base_topo + goff * K
            p_off = base_prot + goff * (K * 9)
            q_off = base_ptr + goff * (K * 3)
            for k in range(K):
                ti = plsc.load_gather(topo_v, [t_off + k])
                w = plsc.load_gather(conf_v, [t_off + k])
                t9 = ti * 9
                t3 = ti * 3
                Rj = [plsc.load_gather(rot_v, [t9 + m]) for m in range(9)]
                tj = [plsc.load_gather(trans_v, [t3 + m]) for m in range(3)]
                pm = [plsc.load_gather(prot_v, [p_off + (k * 9 + m)])
                      for m in range(9)]
                qm = [plsc.load_gather(ptrans_v, [q_off + (k * 3 + m)])
                      for m in range(3)]
                for r in range(3):
                    a0, a1, a2 = Rj[3 * r], Rj[3 * r + 1], Rj[3 * r + 2]
                    for c in range(3):
                        comp = a0 * pm[c] + a1 * pm[3 + c] + a2 * pm[6 + c]
                        acc_r[3 * r + c] = acc_r[3 * r + c] + w * comp
                    ct = a0 * qm[0] + a1 * qm[1] + a2 * qm[2] + tj[r]
                    acc_t[r] = acc_t[r] + w * ct
                acc_w = acc_w + w
            col = ch * NCH + goff
            for m in range(9):
                out_v[pl.ds(m * NPW + col, 16)] = acc_r[m]
            for m in range(3):
                out_v[pl.ds((9 + m) * NPW + col, 16)] = acc_t[m]
            out_v[pl.ds(12 * NPW + col, 16)] = acc_w
            return 0

        lax.fori_loop(0, GROUPS, group_body, 0)
        return 0

    lax.fori_loop(0, CHUNKS, chunk_body, 0)
    gbase = b * N + node0
    for ci in range(13):
        pltpu.sync_copy(out_v.at[pl.ds(ci * NPW, NPW)],
                        out_h.at[pl.ds(ci * (B * N) + gbase, NPW)])


@functools.lru_cache(maxsize=1)
def _sc_call():
    return pl.kernel(
        _sc_body,
        out_type=jax.ShapeDtypeStruct((13 * B * N,), jnp.float32),
        mesh=plsc.VectorSubcoreMesh(core_axis_name="c", subcore_axis_name="s",
                                    num_cores=NC, num_subcores=NS),
        scratch_types=[
            pltpu.VMEM((N * 9,), jnp.float32),
            pltpu.VMEM((N * 3,), jnp.float32),
            pltpu.VMEM((NCH * K,), jnp.int32),
            pltpu.VMEM((NCH * K,), jnp.float32),
            pltpu.VMEM((NCH * K * 3,), jnp.float32),
            pltpu.VMEM((NCH * K * 9,), jnp.float32),
            pltpu.VMEM((13 * NPW,), jnp.float32),
        ],
        compiler_params=pltpu.CompilerParams(needs_layout_passes=False),
    )


def _svd_body(acc_ref, out_ref):
    x = acc_ref[...]
    wsum = x[12:13, :]
    inv = 1.0 / wsum
    m = [[x[3 * r + c:3 * r + c + 1, :] * inv for c in range(3)]
         for r in range(3)]
    tr = [x[9 + r:10 + r, :] * inv for r in range(3)]

    a = [[sum(m[r][i] * m[r][j] for r in range(3)) for j in range(3)]
         for i in range(3)]
    one = jnp.ones_like(a[0][0])
    zero = jnp.zeros_like(a[0][0])
    v = [[one if i == j else zero for j in range(3)] for i in range(3)]
    for _ in range(4):
        for (p, q) in ((0, 1), (0, 2), (1, 2)):
            apq = a[p][q]
            small = jnp.abs(apq) < 1e-30
            tau = (a[q][q] - a[p][p]) / jnp.where(small, 1.0, 2.0 * apq)
            t = jnp.sign(tau) / (jnp.abs(tau) + jnp.sqrt(1.0 + tau * tau))
            t = jnp.where(small, 0.0, t)
            c = 1.0 / jnp.sqrt(1.0 + t * t)
            s = t * c
            for r in range(3):
                arp, arq = a[r][p], a[r][q]
                a[r][p] = c * arp - s * arq
                a[r][q] = s * arp + c * arq
            for ci in range(3):
                apc, aqc = a[p][ci], a[q][ci]
                a[p][ci] = c * apc - s * aqc
                a[q][ci] = s * apc + c * aqc
            for r in range(3):
                vrp, vrq = v[r][p], v[r][q]
                v[r][p] = c * vrp - s * vrq
                v[r][q] = s * vrp + c * vrq

    lam = [a[0][0], a[1][1], a[2][2]]

    def cswap(i, j):
        cond = lam[i] < lam[j]
        lam[i], lam[j] = (jnp.where(cond, lam[j], lam[i]),
                          jnp.where(cond, lam[i], lam[j]))
        for r in range(3):
            v[r][i], v[r][j] = (jnp.where(cond, v[r][j], v[r][i]),
                                jnp.where(cond, v[r][i], v[r][j]))

    cswap(0, 1)
    cswap(0, 2)
    cswap(1, 2)
    detv = (v[0][0] * (v[1][1] * v[2][2] - v[1][2] * v[2][1])
            - v[0][1] * (v[1][0] * v[2][2] - v[1][2] * v[2][0])
            + v[0][2] * (v[1][0] * v[2][1] - v[1][1] * v[2][0]))
    sgn = jnp.where(detv < 0, -1.0, 1.0)
    u1 = [sum(m[r][c] * v[c][0] for c in range(3)) for r in range(3)]
    u2 = [sum(m[r][c] * v[c][1] for c in range(3)) for r in range(3)]
    n1 = jax.lax.rsqrt(u1[0] * u1[0] + u1[1] * u1[1] + u1[2] * u1[2])
    n2 = jax.lax.rsqrt(u2[0] * u2[0] + u2[1] * u2[1] + u2[2] * u2[2])
    u1 = [e * n1 for e in u1]
    u2 = [e * n2 for e in u2]
    u3 = [sgn * (u1[1] * u2[2] - u1[2] * u2[1]),
          sgn * (u1[2] * u2[0] - u1[0] * u2[2]),
          sgn * (u1[0] * u2[1] - u1[1] * u2[0])]
    rows = [u1[r] * v[c][0] + u2[r] * v[c][1] + u3[r] * v[c][2]
            for r in range(3) for c in range(3)]
    rows.extend(tr)
    out_ref[...] = jnp.concatenate(rows, axis=0)


_TC_BLK = 2048
_svd_call = pl.pallas_call(
    _svd_body,
    out_shape=jax.ShapeDtypeStruct((12, B * N), jnp.float32),
    grid=((B * N) // _TC_BLK,),
    in_specs=[pl.BlockSpec((13, _TC_BLK), lambda i: (0, i))],
    out_specs=pl.BlockSpec((12, _TC_BLK), lambda i: (0, i)),
)


def kernel(rot, trans, pair_rot, pair_trans, confidences, topology):
    rot_f = rot.reshape(-1)
    trans_f = trans.reshape(-1)
    prot_f = pair_rot.reshape(-1)
    ptrans_f = pair_trans.reshape(-1)
    conf_f = confidences.reshape(-1)
    topo_f = topology.reshape(-1).astype(jnp.int32)
    acc = _sc_call()(rot_f, trans_f, prot_f, ptrans_f, conf_f, topo_f)
    acc = acc.reshape(13, B * N)
    out = _svd_call(acc)
    out_rot = out[:9].reshape(3, 3, B, N).transpose(2, 3, 0, 1)
    out_trans = out[9:12].reshape(3, B, N).transpose(1, 2, 0)
    return out_rot, out_trans


# speedup vs baseline: 1009.4982x; 31.6584x over previous
import functools

import jax
import jax.numpy as jnp
from jax import lax
from jax.experimental import pallas as pl
from jax.experimental.pallas import tpu as pltpu
from jax.experimental.pallas import tpu_sc as plsc

B, N, K = 8, 4096, 32
NC, NS = 2, 16
NW = NC * NS
NPW = (B * N) // NW
WPB = N // NPW
NT = N // 128
TPW = NPW // 128


def _sc_body(rot_h, trans_h, prot_h, ptrans_h, conf_h, topo_h, out_h,
             rot_v, trans_v, topo_v, conf_v, ptrans_v, prot_v, out_v):
    wid = lax.axis_index("s") * NC + lax.axis_index("c")
    b = wid // WPB
    nt0 = (wid % WPB) * TPW

    pltpu.sync_copy(rot_h.at[:, :, pl.ds(b, 1), :], rot_v)
    pltpu.sync_copy(trans_h.at[:, :, pl.ds(b, 1), :], trans_v)

    zero16 = jnp.zeros((16,), jnp.int32)
    splat9 = [jnp.full((16,), m, jnp.int32) for m in range(9)]

    def chunk_body(ch, _):
        nt = nt0 + ch
        pltpu.sync_copy(topo_h.at[pl.ds(b * 4, 4), pl.ds(nt, 1)], topo_v)
        pltpu.sync_copy(conf_h.at[pl.ds(b * K, K), pl.ds(nt, 1)], conf_v)
        pltpu.sync_copy(ptrans_h.at[pl.ds(b * 12, 12), pl.ds(nt, 1)],
                        ptrans_v)
        pltpu.sync_copy(prot_h.at[pl.ds(b * 36, 36), pl.ds(nt, 1)], prot_v)

        def group_body(g, _):
            off = g * 16
            zero = jnp.zeros((16,), jnp.float32)
            acc_r = [zero] * 9
            acc_t = [zero] * 3
            acc_w = zero
            for k in range(K):
                kt, kr = k >> 3, k & 7
                ti = topo_v[kt, 0, kr, pl.ds(off, 16)]
                w = conf_v[k, 0, pl.ds(off, 16)]
                hi = ti >> 7
                lo = ti & 127
                Rj = [plsc.load_gather(rot_v, [splat9[m], hi, zero16, lo])
                      for m in range(9)]
                tj = [plsc.load_gather(trans_v, [splat9[m], hi, zero16, lo])
                      for m in range(3)]
                pm = [prot_v[m * 4 + kt, 0, kr, pl.ds(off, 16)]
                      for m in range(9)]
                qm = [ptrans_v[m * 4 + kt, 0, kr, pl.ds(off, 16)]
                      for m in range(3)]
                for r in range(3):
                    a0, a1, a2 = Rj[3 * r], Rj[3 * r + 1], Rj[3 * r + 2]
                    for c in range(3):
                        comp = a0 * pm[c] + a1 * pm[3 + c] + a2 * pm[6 + c]
                        acc_r[3 * r + c] = acc_r[3 * r + c] + w * comp
                    ct = a0 * qm[0] + a1 * qm[1] + a2 * qm[2] + tj[r]
                    acc_t[r] = acc_t[r] + w * ct
                acc_w = acc_w + w
            col = ch * 128 + off
            for m in range(9):
                out_v[pl.ds(m * NPW + col, 16)] = acc_r[m]
            for m in range(3):
                out_v[pl.ds((9 + m) * NPW + col, 16)] = acc_t[m]
            out_v[pl.ds(12 * NPW + col, 16)] = acc_w
            return 0

        lax.fori_loop(0, 8, group_body, 0)
        return 0

    lax.fori_loop(0, TPW, chunk_body, 0)
    gbase = b * N + nt0 * 128
    for ci in range(13):
        pltpu.sync_copy(out_v.at[pl.ds(ci * NPW, NPW)],
                        out_h.at[pl.ds(ci * (B * N) + gbase, NPW)])


@functools.lru_cache(maxsize=1)
def _sc_call():
    return pl.kernel(
        _sc_body,
        out_type=jax.ShapeDtypeStruct((13 * B * N,), jnp.float32),
        mesh=plsc.VectorSubcoreMesh(core_axis_name="c", subcore_axis_name="s",
                                    num_cores=NC, num_subcores=NS),
        scratch_types=[
            pltpu.VMEM((9, NT, 1, 128), jnp.float32),
            pltpu.VMEM((3, NT, 1, 128), jnp.float32),
            pltpu.VMEM((4, 1, 8, 128), jnp.int32),
            pltpu.VMEM((K, 1, 128), jnp.float32),
            pltpu.VMEM((12, 1, 8, 128), jnp.float32),
            pltpu.VMEM((36, 1, 8, 128), jnp.float32),
            pltpu.VMEM((13 * NPW,), jnp.float32),
        ],
        compiler_params=pltpu.CompilerParams(needs_layout_passes=False),
    )


def _svd_body(acc_ref, out_ref):
    x = acc_ref[...]
    wsum = x[12:13, :]
    inv = 1.0 / wsum
    m = [[x[3 * r + c:3 * r + c + 1, :] * inv for c in range(3)]
         for r in range(3)]
    tr = [x[9 + r:10 + r, :] * inv for r in range(3)]

    a = [[sum(m[r][i] * m[r][j] for r in range(3)) for j in range(3)]
         for i in range(3)]
    one = jnp.ones_like(a[0][0])
    zero = jnp.zeros_like(a[0][0])
    v = [[one if i == j else zero for j in range(3)] for i in range(3)]
    for _ in range(4):
        for (p, q) in ((0, 1), (0, 2), (1, 2)):
            apq = a[p][q]
            small = jnp.abs(apq) < 1e-30
            tau = (a[q][q] - a[p][p]) / jnp.where(small, 1.0, 2.0 * apq)
            t = jnp.sign(tau) / (jnp.abs(tau) + jnp.sqrt(1.0 + tau * tau))
            t = jnp.where(small, 0.0, t)
            c = 1.0 / jnp.sqrt(1.0 + t * t)
            s = t * c
            for r in range(3):
                arp, arq = a[r][p], a[r][q]
                a[r][p] = c * arp - s * arq
                a[r][q] = s * arp + c * arq
            for ci in range(3):
                apc, aqc = a[p][ci], a[q][ci]
                a[p][ci] = c * apc - s * aqc
                a[q][ci] = s * apc + c * aqc
            for r in range(3):
                vrp, vrq = v[r][p], v[r][q]
                v[r][p] = c * vrp - s * vrq
                v[r][q] = s * vrp + c * vrq

    lam = [a[0][0], a[1][1], a[2][2]]

    def cswap(i, j):
        cond = lam[i] < lam[j]
        lam[i], lam[j] = (jnp.where(cond, lam[j], lam[i]),
                          jnp.where(cond, lam[i], lam[j]))
        for r in range(3):
            v[r][i], v[r][j] = (jnp.where(cond, v[r][j], v[r][i]),
                                jnp.where(cond, v[r][i], v[r][j]))

    cswap(0, 1)
    cswap(0, 2)
    cswap(1, 2)
    detv = (v[0][0] * (v[1][1] * v[2][2] - v[1][2] * v[2][1])
            - v[0][1] * (v[1][0] * v[2][2] - v[1][2] * v[2][0])
            + v[0][2] * (v[1][0] * v[2][1] - v[1][1] * v[2][0]))
    sgn = jnp.where(detv < 0, -1.0, 1.0)
    u1 = [sum(m[r][c] * v[c][0] for c in range(3)) for r in range(3)]
    u2 = [sum(m[r][c] * v[c][1] for c in range(3)) for r in range(3)]
    n1 = jax.lax.rsqrt(u1[0] * u1[0] + u1[1] * u1[1] + u1[2] * u1[2])
    n2 = jax.lax.rsqrt(u2[0] * u2[0] + u2[1] * u2[1] + u2[2] * u2[2])
    u1 = [e * n1 for e in u1]
    u2 = [e * n2 for e in u2]
    u3 = [sgn * (u1[1] * u2[2] - u1[2] * u2[1]),
          sgn * (u1[2] * u2[0] - u1[0] * u2[2]),
          sgn * (u1[0] * u2[1] - u1[1] * u2[0])]
    rows = [u1[r] * v[c][0] + u2[r] * v[c][1] + u3[r] * v[c][2]
            for r in range(3) for c in range(3)]
    rows.extend(tr)
    out_ref[...] = jnp.concatenate(rows, axis=0)


_TC_BLK = 2048
_svd_call = pl.pallas_call(
    _svd_body,
    out_shape=jax.ShapeDtypeStruct((12, B * N), jnp.float32),
    grid=((B * N) // _TC_BLK,),
    in_specs=[pl.BlockSpec((13, _TC_BLK), lambda i: (0, i))],
    out_specs=pl.BlockSpec((12, _TC_BLK), lambda i: (0, i)),
)


def kernel(rot, trans, pair_rot, pair_trans, confidences, topology):
    rot_f = (rot.transpose(2, 3, 0, 1).reshape(9, B, NT, 128)
             .transpose(0, 2, 1, 3))
    trans_f = (trans.transpose(2, 0, 1).reshape(3, B, NT, 128)
               .transpose(0, 2, 1, 3))
    prot_f = (pair_rot.transpose(0, 3, 4, 2, 1)
              .reshape(B, 3, 3, 4, 8, NT, 128)
              .transpose(0, 1, 2, 3, 5, 4, 6).reshape(B * 36, NT, 8, 128))
    ptrans_f = (pair_trans.transpose(0, 3, 2, 1).reshape(B, 3, 4, 8, NT, 128)
                .transpose(0, 1, 2, 4, 3, 5).reshape(B * 12, NT, 8, 128))
    conf_f = (confidences.reshape(B, N, K).transpose(0, 2, 1)
              .reshape(B * K, NT, 128))
    topo_f = (topology.astype(jnp.int32).transpose(0, 2, 1)
              .reshape(B, 4, 8, NT, 128).transpose(0, 1, 3, 2, 4)
              .reshape(B * 4, NT, 8, 128))
    acc = _sc_call()(rot_f, trans_f, prot_f, ptrans_f, conf_f, topo_f)
    out = _svd_call(acc.reshape(13, B * N))
    out_rot = out[:9].reshape(3, 3, B, N).transpose(2, 3, 0, 1)
    out_trans = out[9:12].reshape(3, B, N).transpose(1, 2, 0)
    return out_rot, out_trans


# speedup vs baseline: 1447.3089x; 1.4337x over previous
import functools

import jax
import jax.numpy as jnp
from jax import lax
from jax.experimental import pallas as pl
from jax.experimental.pallas import tpu as pltpu
from jax.experimental.pallas import tpu_sc as plsc

B, N, K = 8, 4096, 32
NC, NS = 2, 16
NW = NC * NS
NPW = (B * N) // NW
WPB = N // NPW
NT = N // 128
TPW = NPW // 128


def _sc_body(rot_h, trans_h, prot_h, ptrans_h, conf_h, topo_h, out_h,
             rot_v, trans_v, topo_v, conf_v, ptrans_v, prot_v, out_v):
    wid = lax.axis_index("s") * NC + lax.axis_index("c")
    b = wid // WPB
    nt0 = (wid % WPB) * TPW

    pltpu.sync_copy(rot_h.at[:, :, pl.ds(b, 1), :], rot_v)
    pltpu.sync_copy(trans_h.at[:, :, pl.ds(b, 1), :], trans_v)

    zero16 = jnp.zeros((16,), jnp.int32)
    splat9 = [jnp.full((16,), m, jnp.int32) for m in range(9)]

    def chunk_body(ch, _):
        nt = nt0 + ch
        pltpu.sync_copy(topo_h.at[pl.ds(b * 4, 4), pl.ds(nt, 1)], topo_v)
        pltpu.sync_copy(conf_h.at[pl.ds(b * K, K), pl.ds(nt, 1)], conf_v)
        pltpu.sync_copy(ptrans_h.at[pl.ds(b * 12, 12), pl.ds(nt, 1)],
                        ptrans_v)
        pltpu.sync_copy(prot_h.at[pl.ds(b * 36, 36), pl.ds(nt, 1)], prot_v)

        def group_body(g, _):
            off = g * 16
            zero = jnp.zeros((16,), jnp.float32)
            acc_r = [zero] * 9
            acc_t = [zero] * 3
            acc_w = zero
            for k in range(K):
                kt, kr = k >> 3, k & 7
                ti = topo_v[kt, 0, kr, pl.ds(off, 16)]
                w = conf_v[k, 0, pl.ds(off, 16)]
                hi = ti >> 7
                lo = ti & 127
                Rj = [plsc.load_gather(rot_v, [splat9[m], hi, zero16, lo])
                      for m in range(9)]
                tj = [plsc.load_gather(trans_v, [splat9[m], hi, zero16, lo])
                      for m in range(3)]
                pm = [prot_v[m * 4 + kt, 0, kr, pl.ds(off, 16)]
                      for m in range(9)]
                qm = [ptrans_v[m * 4 + kt, 0, kr, pl.ds(off, 16)]
                      for m in range(3)]
                for r in range(3):
                    a0, a1, a2 = Rj[3 * r], Rj[3 * r + 1], Rj[3 * r + 2]
                    for c in range(3):
                        comp = a0 * pm[c] + a1 * pm[3 + c] + a2 * pm[6 + c]
                        acc_r[3 * r + c] = acc_r[3 * r + c] + w * comp
                    ct = a0 * qm[0] + a1 * qm[1] + a2 * qm[2] + tj[r]
                    acc_t[r] = acc_t[r] + w * ct
                acc_w = acc_w + w
            col = ch * 128 + off
            for m in range(9):
                out_v[pl.ds(m * NPW + col, 16)] = acc_r[m]
            for m in range(3):
                out_v[pl.ds((9 + m) * NPW + col, 16)] = acc_t[m]
            out_v[pl.ds(12 * NPW + col, 16)] = acc_w
            return 0

        lax.fori_loop(0, 8, group_body, 0)
        return 0

    lax.fori_loop(0, TPW, chunk_body, 0)
    gbase = b * N + nt0 * 128
    for ci in range(13):
        pltpu.sync_copy(out_v.at[pl.ds(ci * NPW, NPW)],
                        out_h.at[pl.ds(ci * (B * N) + gbase, NPW)])


@functools.lru_cache(maxsize=1)
def _sc_call():
    return pl.kernel(
        _sc_body,
        out_type=jax.ShapeDtypeStruct((13 * B * N,), jnp.float32),
        mesh=plsc.VectorSubcoreMesh(core_axis_name="c", subcore_axis_name="s",
                                    num_cores=NC, num_subcores=NS),
        scratch_types=[
            pltpu.VMEM((9, NT, 1, 128), jnp.float32),
            pltpu.VMEM((3, NT, 1, 128), jnp.float32),
            pltpu.VMEM((4, 1, 8, 128), jnp.int32),
            pltpu.VMEM((K, 1, 128), jnp.float32),
            pltpu.VMEM((12, 1, 8, 128), jnp.float32),
            pltpu.VMEM((36, 1, 8, 128), jnp.float32),
            pltpu.VMEM((13 * NPW,), jnp.float32),
        ],
        compiler_params=pltpu.CompilerParams(needs_layout_passes=False),
    )


def _svd_body(acc_ref, out_ref):
    x = acc_ref[...]
    wsum = x[12]
    inv = 1.0 / wsum
    m = [[x[3 * r + c] * inv for c in range(3)] for r in range(3)]
    tr = [x[9 + r] * inv for r in range(3)]

    a = [[sum(m[r][i] * m[r][j] for r in range(3)) for j in range(3)]
         for i in range(3)]
    one = jnp.ones_like(a[0][0])
    zero = jnp.zeros_like(a[0][0])
    v = [[one if i == j else zero for j in range(3)] for i in range(3)]
    for _ in range(4):
        for (p, q) in ((0, 1), (0, 2), (1, 2)):
            apq = a[p][q]
            small = jnp.abs(apq) < 1e-30
            tau = (a[q][q] - a[p][p]) / jnp.where(small, 1.0, 2.0 * apq)
            t = jnp.sign(tau) / (jnp.abs(tau) + jnp.sqrt(1.0 + tau * tau))
            t = jnp.where(small, 0.0, t)
            c = 1.0 / jnp.sqrt(1.0 + t * t)
            s = t * c
            for r in range(3):
                arp, arq = a[r][p], a[r][q]
                a[r][p] = c * arp - s * arq
                a[r][q] = s * arp + c * arq
            for ci in range(3):
                apc, aqc = a[p][ci], a[q][ci]
                a[p][ci] = c * apc - s * aqc
                a[q][ci] = s * apc + c * aqc
            for r in range(3):
                vrp, vrq = v[r][p], v[r][q]
                v[r][p] = c * vrp - s * vrq
                v[r][q] = s * vrp + c * vrq

    lam = [a[0][0], a[1][1], a[2][2]]

    def cswap(i, j):
        cond = lam[i] < lam[j]
        lam[i], lam[j] = (jnp.where(cond, lam[j], lam[i]),
                          jnp.where(cond, lam[i], lam[j]))
        for r in range(3):
            v[r][i], v[r][j] = (jnp.where(cond, v[r][j], v[r][i]),
                                jnp.where(cond, v[r][i], v[r][j]))

    cswap(0, 1)
    cswap(0, 2)
    cswap(1, 2)
    detv = (v[0][0] * (v[1][1] * v[2][2] - v[1][2] * v[2][1])
            - v[0][1] * (v[1][0] * v[2][2] - v[1][2] * v[2][0])
            + v[0][2] * (v[1][0] * v[2][1] - v[1][1] * v[2][0]))
    sgn = jnp.where(detv < 0, -1.0, 1.0)
    u1 = [sum(m[r][c] * v[c][0] for c in range(3)) for r in range(3)]
    u2 = [sum(m[r][c] * v[c][1] for c in range(3)) for r in range(3)]
    n1 = jax.lax.rsqrt(u1[0] * u1[0] + u1[1] * u1[1] + u1[2] * u1[2])
    n2 = jax.lax.rsqrt(u2[0] * u2[0] + u2[1] * u2[1] + u2[2] * u2[2])
    u1 = [e * n1 for e in u1]
    u2 = [e * n2 for e in u2]
    u3 = [sgn * (u1[1] * u2[2] - u1[2] * u2[1]),
          sgn * (u1[2] * u2[0] - u1[0] * u2[2]),
          sgn * (u1[0] * u2[1] - u1[1] * u2[0])]
    rows = [u1[r] * v[c][0] + u2[r] * v[c][1] + u3[r] * v[c][2]
            for r in range(3) for c in range(3)]
    rows.extend(tr)
    out_ref[...] = jnp.stack(rows, axis=0)


_TC_SUB = 32
_svd_call = pl.pallas_call(
    _svd_body,
    out_shape=jax.ShapeDtypeStruct((12, (B * N) // (_TC_SUB * 128),
                                    _TC_SUB, 128), jnp.float32),
    grid=((B * N) // (_TC_SUB * 128),),
    in_specs=[pl.BlockSpec((13, 1, _TC_SUB, 128), lambda i: (0, i, 0, 0))],
    out_specs=pl.BlockSpec((12, 1, _TC_SUB, 128), lambda i: (0, i, 0, 0)),
)


def kernel(rot, trans, pair_rot, pair_trans, confidences, topology):
    rot_f = (rot.transpose(2, 3, 0, 1).reshape(9, B, NT, 128)
             .transpose(0, 2, 1, 3))
    trans_f = (trans.transpose(2, 0, 1).reshape(3, B, NT, 128)
               .transpose(0, 2, 1, 3))
    prot_f = (pair_rot.transpose(0, 3, 4, 2, 1)
              .reshape(B, 3, 3, 4, 8, NT, 128)
              .transpose(0, 1, 2, 3, 5, 4, 6).reshape(B * 36, NT, 8, 128))
    ptrans_f = (pair_trans.transpose(0, 3, 2, 1).reshape(B, 3, 4, 8, NT, 128)
                .transpose(0, 1, 2, 4, 3, 5).reshape(B * 12, NT, 8, 128))
    conf_f = (confidences.transpose(0, 2, 3, 1)
              .reshape(B * K, NT, 128))
    topo_f = (topology.astype(jnp.int32).transpose(0, 2, 1)
              .reshape(B, 4, 8, NT, 128).transpose(0, 1, 3, 2, 4)
              .reshape(B * 4, NT, 8, 128))
    acc = _sc_call()(rot_f, trans_f, prot_f, ptrans_f, conf_f, topo_f)
    out = _svd_call(acc.reshape(13, (B * N) // (_TC_SUB * 128), _TC_SUB, 128))
    out = out.reshape(12, B * N)
    out_rot = out[:9].reshape(3, 3, B, N).transpose(2, 3, 0, 1)
    out_trans = out[9:12].reshape(3, B, N).transpose(1, 2, 0)
    return out_rot, out_trans


# speedup vs baseline: 1449.2349x; 1.0013x over previous
import functools

import jax
import jax.numpy as jnp
from jax import lax
from jax.experimental import pallas as pl
from jax.experimental.pallas import tpu as pltpu
from jax.experimental.pallas import tpu_sc as plsc

B, N, K = 8, 4096, 32
NC, NS = 2, 16
NW = NC * NS
NPW = (B * N) // NW
WPB = N // NPW
NT = N // 128
TPW = NPW // 128


def _sc_body(rot_h, trans_h, prot_h, ptrans_h, conf_h, topo_h, out_h,
             rot_v, trans_v, topo_v, conf_v, ptrans_v, prot_v, out_v):
    wid = lax.axis_index("s") * NC + lax.axis_index("c")
    b = wid // WPB
    nt0 = (wid % WPB) * TPW

    pltpu.sync_copy(rot_h.at[:, :, pl.ds(b, 1), :], rot_v)
    pltpu.sync_copy(trans_h.at[:, :, pl.ds(b, 1), :], trans_v)

    zero16 = jnp.zeros((16,), jnp.int32)

    def chunk_body(ch, _):
        nt = nt0 + ch
        pltpu.sync_copy(topo_h.at[pl.ds(b * 4, 4), pl.ds(nt, 1)], topo_v)
        pltpu.sync_copy(conf_h.at[pl.ds(b * K, K), pl.ds(nt, 1)], conf_v)
        pltpu.sync_copy(ptrans_h.at[pl.ds(b * 12, 12), pl.ds(nt, 1)],
                        ptrans_v)
        pltpu.sync_copy(prot_h.at[pl.ds(b * 36, 36), pl.ds(nt, 1)], prot_v)

        @plsc.parallel_loop(0, 8)
        def group_body(g):
            off = g * 16
            zero = jnp.zeros((16,), jnp.float32)
            acc_r = [zero] * 9
            acc_t = [zero] * 3
            acc_w = zero
            for k in range(K):
                kt, kr = k >> 3, k & 7
                ti = topo_v[kt, 0, kr, pl.ds(off, 16)]
                w = conf_v[k, 0, pl.ds(off, 16)]
                Rj = [plsc.load_gather(rot_v,
                                       [zero16, zero16, zero16, ti + m * N])
                      for m in range(9)]
                tj = [plsc.load_gather(trans_v,
                                       [zero16, zero16, zero16, ti + m * N])
                      for m in range(3)]
                pm = [prot_v[m * 4 + kt, 0, kr, pl.ds(off, 16)]
                      for m in range(9)]
                qm = [ptrans_v[m * 4 + kt, 0, kr, pl.ds(off, 16)]
                      for m in range(3)]
                for r in range(3):
                    a0, a1, a2 = Rj[3 * r], Rj[3 * r + 1], Rj[3 * r + 2]
                    for c in range(3):
                        comp = a0 * pm[c] + a1 * pm[3 + c] + a2 * pm[6 + c]
                        acc_r[3 * r + c] = acc_r[3 * r + c] + w * comp
                    ct = a0 * qm[0] + a1 * qm[1] + a2 * qm[2] + tj[r]
                    acc_t[r] = acc_t[r] + w * ct
                acc_w = acc_w + w
            col = ch * 128 + off
            for m in range(9):
                out_v[pl.ds(m * NPW + col, 16)] = acc_r[m]
            for m in range(3):
                out_v[pl.ds((9 + m) * NPW + col, 16)] = acc_t[m]
            out_v[pl.ds(12 * NPW + col, 16)] = acc_w

        return 0

    lax.fori_loop(0, TPW, chunk_body, 0)
    gbase = b * N + nt0 * 128
    for ci in range(13):
        pltpu.sync_copy(out_v.at[pl.ds(ci * NPW, NPW)],
                        out_h.at[pl.ds(ci * (B * N) + gbase, NPW)])


@functools.lru_cache(maxsize=1)
def _sc_call():
    return pl.kernel(
        _sc_body,
        out_type=jax.ShapeDtypeStruct((13 * B * N,), jnp.float32),
        mesh=plsc.VectorSubcoreMesh(core_axis_name="c", subcore_axis_name="s",
                                    num_cores=NC, num_subcores=NS),
        scratch_types=[
            pltpu.VMEM((9, NT, 1, 128), jnp.float32),
            pltpu.VMEM((3, NT, 1, 128), jnp.float32),
            pltpu.VMEM((4, 1, 8, 128), jnp.int32),
            pltpu.VMEM((K, 1, 128), jnp.float32),
            pltpu.VMEM((12, 1, 8, 128), jnp.float32),
            pltpu.VMEM((36, 1, 8, 128), jnp.float32),
            pltpu.VMEM((13 * NPW,), jnp.float32),
        ],
        compiler_params=pltpu.CompilerParams(needs_layout_passes=False),
    )


def _svd_body(acc_ref, out_ref):
    x = acc_ref[...]
    wsum = x[12]
    inv = 1.0 / wsum
    m = [[x[3 * r + c] * inv for c in range(3)] for r in range(3)]
    tr = [x[9 + r] * inv for r in range(3)]

    a = [[sum(m[r][i] * m[r][j] for r in range(3)) for j in range(3)]
         for i in range(3)]
    one = jnp.ones_like(a[0][0])
    zero = jnp.zeros_like(a[0][0])
    v = [[one if i == j else zero for j in range(3)] for i in range(3)]
    for _ in range(4):
        for (p, q) in ((0, 1), (0, 2), (1, 2)):
            apq = a[p][q]
            small = jnp.abs(apq) < 1e-30
            tau = (a[q][q] - a[p][p]) / jnp.where(small, 1.0, 2.0 * apq)
            t = jnp.sign(tau) / (jnp.abs(tau) + jnp.sqrt(1.0 + tau * tau))
            t = jnp.where(small, 0.0, t)
            c = 1.0 / jnp.sqrt(1.0 + t * t)
            s = t * c
            for r in range(3):
                arp, arq = a[r][p], a[r][q]
                a[r][p] = c * arp - s * arq
                a[r][q] = s * arp + c * arq
            for ci in range(3):
                apc, aqc = a[p][ci], a[q][ci]
                a[p][ci] = c * apc - s * aqc
                a[q][ci] = s * apc + c * aqc
            for r in range(3):
                vrp, vrq = v[r][p], v[r][q]
                v[r][p] = c * vrp - s * vrq
                v[r][q] = s * vrp + c * vrq

    lam = [a[0][0], a[1][1], a[2][2]]

    def cswap(i, j):
        cond = lam[i] < lam[j]
        lam[i], lam[j] = (jnp.where(cond, lam[j], lam[i]),
                          jnp.where(cond, lam[i], lam[j]))
        for r in range(3):
            v[r][i], v[r][j] = (jnp.where(cond, v[r][j], v[r][i]),
                                jnp.where(cond, v[r][i], v[r][j]))

    cswap(0, 1)
    cswap(0, 2)
    cswap(1, 2)
    detv = (v[0][0] * (v[1][1] * v[2][2] - v[1][2] * v[2][1])
            - v[0][1] * (v[1][0] * v[2][2] - v[1][2] * v[2][0])
            + v[0][2] * (v[1][0] * v[2][1] - v[1][1] * v[2][0]))
    sgn = jnp.where(detv < 0, -1.0, 1.0)
    u1 = [sum(m[r][c] * v[c][0] for c in range(3)) for r in range(3)]
    u2 = [sum(m[r][c] * v[c][1] for c in range(3)) for r in range(3)]
    n1 = jax.lax.rsqrt(u1[0] * u1[0] + u1[1] * u1[1] + u1[2] * u1[2])
    n2 = jax.lax.rsqrt(u2[0] * u2[0] + u2[1] * u2[1] + u2[2] * u2[2])
    u1 = [e * n1 for e in u1]
    u2 = [e * n2 for e in u2]
    u3 = [sgn * (u1[1] * u2[2] - u1[2] * u2[1]),
          sgn * (u1[2] * u2[0] - u1[0] * u2[2]),
          sgn * (u1[0] * u2[1] - u1[1] * u2[0])]
    rows = [u1[r] * v[c][0] + u2[r] * v[c][1] + u3[r] * v[c][2]
            for r in range(3) for c in range(3)]
    rows.extend(tr)
    out_ref[...] = jnp.stack(rows, axis=0)


_TC_SUB = 32
_svd_call = pl.pallas_call(
    _svd_body,
    out_shape=jax.ShapeDtypeStruct((12, (B * N) // (_TC_SUB * 128),
                                    _TC_SUB, 128), jnp.float32),
    grid=((B * N) // (_TC_SUB * 128),),
    in_specs=[pl.BlockSpec((13, 1, _TC_SUB, 128), lambda i: (0, i, 0, 0))],
    out_specs=pl.BlockSpec((12, 1, _TC_SUB, 128), lambda i: (0, i, 0, 0)),
)


def kernel(rot, trans, pair_rot, pair_trans, confidences, topology):
    rot_f = (rot.transpose(2, 3, 0, 1).reshape(9, B, NT, 128)
             .transpose(0, 2, 1, 3))
    trans_f = (trans.transpose(2, 0, 1).reshape(3, B, NT, 128)
               .transpose(0, 2, 1, 3))
    prot_f = (pair_rot.transpose(0, 3, 4, 2, 1)
              .reshape(B, 3, 3, 4, 8, NT, 128)
              .transpose(0, 1, 2, 3, 5, 4, 6).reshape(B * 36, NT, 8, 128))
    ptrans_f = (pair_trans.transpose(0, 3, 2, 1).reshape(B, 3, 4, 8, NT, 128)
                .transpose(0, 1, 2, 4, 3, 5).reshape(B * 12, NT, 8, 128))
    conf_f = (confidences.transpose(0, 2, 3, 1)
              .reshape(B * K, NT, 128))
    topo_f = (topology.astype(jnp.int32).transpose(0, 2, 1)
              .reshape(B, 4, 8, NT, 128).transpose(0, 1, 3, 2, 4)
              .reshape(B * 4, NT, 8, 128))
    acc = _sc_call()(rot_f, trans_f, prot_f, ptrans_f, conf_f, topo_f)
    out = _svd_call(acc.reshape(13, (B * N) // (_TC_SUB * 128), _TC_SUB, 128))
    out = out.reshape(12, B * N)
    out_rot = out[:9].reshape(3, 3, B, N).transpose(2, 3, 0, 1)
    out_trans = out[9:12].reshape(3, B, N).transpose(1, 2, 0)
    return out_rot, out_trans


# speedup vs baseline: 2020.3971x; 1.3941x over previous
import functools

import jax
import jax.numpy as jnp
from jax import lax
from jax.experimental import pallas as pl
from jax.experimental.pallas import tpu as pltpu
from jax.experimental.pallas import tpu_sc as plsc

B, N, K = 8, 4096, 32
NC, NS = 2, 16
NW = NC * NS
NPW = (B * N) // NW
WPB = N // NPW
NT = N // 128
TPW = NPW // 128


def _sc_body(rot_h, trans_h, prot_h, ptrans_h, conf_h, topo_h, out_h,
             rot_v, trans_v,
             topo_b0, topo_b1, conf_b0, conf_b1,
             ptrans_b0, ptrans_b1, prot_b0, prot_b1,
             out_v, sem0, sem1, sem2):
    wid = lax.axis_index("s") * NC + lax.axis_index("c")
    b = wid // WPB
    nt0 = (wid % WPB) * TPW

    bufs = ((topo_b0, conf_b0, ptrans_b0, prot_b0, sem0),
            (topo_b1, conf_b1, ptrans_b1, prot_b1, sem1))

    def half_srcs(nt, h):
        return (topo_h.at[pl.ds(b * 4 + 2 * h, 2), pl.ds(nt, 1)],
                conf_h.at[pl.ds(b * K + 16 * h, 16), pl.ds(nt, 1)],
                ptrans_h.at[pl.ds(b * 3, 3), pl.ds(2 * h, 2), pl.ds(nt, 1)],
                prot_h.at[pl.ds(b * 9, 9), pl.ds(2 * h, 2), pl.ds(nt, 1)])

    def issue(nt, h, slot):
        srcs = half_srcs(nt, h)
        for src, dst in zip(srcs, bufs[slot][:4]):
            pltpu.async_copy(src, dst, bufs[slot][4])

    def drain(nt, h, slot):
        srcs = half_srcs(nt, h)
        for src, dst in zip(srcs, bufs[slot][:4]):
            pltpu.make_async_copy(src, dst, bufs[slot][4]).wait()

    pltpu.async_copy(rot_h.at[:, :, pl.ds(b, 1), :], rot_v, sem2)
    pltpu.async_copy(trans_h.at[:, :, pl.ds(b, 1), :], trans_v, sem2)
    issue(nt0, 0, 0)
    pltpu.make_async_copy(rot_h.at[:, :, pl.ds(b, 1), :], rot_v, sem2).wait()
    pltpu.make_async_copy(trans_h.at[:, :, pl.ds(b, 1), :], trans_v,
                          sem2).wait()

    zero16 = jnp.zeros((16,), jnp.int32)

    def compute_half(ch, h, slot):
        topo_v, conf_v, ptrans_v, prot_v, _ = bufs[slot]

        @plsc.parallel_loop(0, 8)
        def group_body(g):
            off = g * 16
            zero = jnp.zeros((16,), jnp.float32)
            acc_r = [zero] * 9
            acc_t = [zero] * 3
            acc_w = zero
            for kk in range(16):
                kt, kr = kk >> 3, kk & 7
                ti = topo_v[kt, 0, kr, pl.ds(off, 16)]
                w = conf_v[kk, 0, pl.ds(off, 16)]
                Rj = [plsc.load_gather(rot_v,
                                       [zero16, zero16, zero16, ti + m * N])
                      for m in range(9)]
                tj = [plsc.load_gather(trans_v,
                                       [zero16, zero16, zero16, ti + m * N])
                      for m in range(3)]
                pm = [prot_v[m, kt, 0, kr, pl.ds(off, 16)] for m in range(9)]
                qm = [ptrans_v[m, kt, 0, kr, pl.ds(off, 16)]
                      for m in range(3)]
                for r in range(3):
                    a0, a1, a2 = Rj[3 * r], Rj[3 * r + 1], Rj[3 * r + 2]
                    for c in range(3):
                        comp = a0 * pm[c] + a1 * pm[3 + c] + a2 * pm[6 + c]
                        acc_r[3 * r + c] = acc_r[3 * r + c] + w * comp
                    ct = a0 * qm[0] + a1 * qm[1] + a2 * qm[2] + tj[r]
                    acc_t[r] = acc_t[r] + w * ct
                acc_w = acc_w + w
            col = ch * 128 + off
            accs = acc_r + acc_t + [acc_w]
            if h == 0:
                for m in range(13):
                    out_v[pl.ds(m * NPW + col, 16)] = accs[m]
            else:
                for m in range(13):
                    plsc.addupdate(out_v.at[pl.ds(m * NPW + col, 16)],
                                   accs[m])

    def chunk_body(ch, _):
        nt = nt0 + ch
        issue(nt, 1, 1)
        drain(nt, 0, 0)
        compute_half(ch, 0, 0)

        @pl.when(ch < TPW - 1)
        def _():
            issue(nt + 1, 0, 0)

        drain(nt, 1, 1)
        compute_half(ch, 1, 1)
        return 0

    lax.fori_loop(0, TPW, chunk_body, 0)
    gbase = b * N + nt0 * 128
    for ci in range(13):
        pltpu.async_copy(out_v.at[pl.ds(ci * NPW, NPW)],
                         out_h.at[pl.ds(ci * (B * N) + gbase, NPW)], sem2)
    for ci in range(13):
        pltpu.make_async_copy(out_v.at[pl.ds(ci * NPW, NPW)],
                              out_h.at[pl.ds(ci * (B * N) + gbase, NPW)],
                              sem2).wait()


@functools.lru_cache(maxsize=1)
def _sc_call():
    half_bufs = [
        pltpu.VMEM((2, 1, 8, 128), jnp.int32),
        pltpu.VMEM((16, 1, 128), jnp.float32),
        pltpu.VMEM((3, 2, 1, 8, 128), jnp.float32),
        pltpu.VMEM((9, 2, 1, 8, 128), jnp.float32),
    ]
    return pl.kernel(
        _sc_body,
        out_type=jax.ShapeDtypeStruct((13 * B * N,), jnp.float32),
        mesh=plsc.VectorSubcoreMesh(core_axis_name="c", subcore_axis_name="s",
                                    num_cores=NC, num_subcores=NS),
        scratch_types=[
            pltpu.VMEM((9, NT, 1, 128), jnp.float32),
            pltpu.VMEM((3, NT, 1, 128), jnp.float32),
            half_bufs[0], half_bufs[0],
            half_bufs[1], half_bufs[1],
            half_bufs[2], half_bufs[2],
            half_bufs[3], half_bufs[3],
            pltpu.VMEM((13 * NPW,), jnp.float32),
            pltpu.SemaphoreType.DMA,
            pltpu.SemaphoreType.DMA,
            pltpu.SemaphoreType.DMA,
        ],
        compiler_params=pltpu.CompilerParams(needs_layout_passes=False),
    )


def _svd_body(acc_ref, out_ref):
    x = acc_ref[...]
    wsum = x[12]
    inv = 1.0 / wsum
    m = [[x[3 * r + c] * inv for c in range(3)] for r in range(3)]
    tr = [x[9 + r] * inv for r in range(3)]

    a = [[sum(m[r][i] * m[r][j] for r in range(3)) for j in range(3)]
         for i in range(3)]
    one = jnp.ones_like(a[0][0])
    zero = jnp.zeros_like(a[0][0])
    v = [[one if i == j else zero for j in range(3)] for i in range(3)]
    for _ in range(4):
        for (p, q) in ((0, 1), (0, 2), (1, 2)):
            apq = a[p][q]
            small = jnp.abs(apq) < 1e-30
            tau = (a[q][q] - a[p][p]) / jnp.where(small, 1.0, 2.0 * apq)
            t = jnp.sign(tau) / (jnp.abs(tau) + jnp.sqrt(1.0 + tau * tau))
            t = jnp.where(small, 0.0, t)
            c = 1.0 / jnp.sqrt(1.0 + t * t)
            s = t * c
            for r in range(3):
                arp, arq = a[r][p], a[r][q]
                a[r][p] = c * arp - s * arq
                a[r][q] = s * arp + c * arq
            for ci in range(3):
                apc, aqc = a[p][ci], a[q][ci]
                a[p][ci] = c * apc - s * aqc
                a[q][ci] = s * apc + c * aqc
            for r in range(3):
                vrp, vrq = v[r][p], v[r][q]
                v[r][p] = c * vrp - s * vrq
                v[r][q] = s * vrp + c * vrq

    lam = [a[0][0], a[1][1], a[2][2]]

    def cswap(i, j):
        cond = lam[i] < lam[j]
        lam[i], lam[j] = (jnp.where(cond, lam[j], lam[i]),
                          jnp.where(cond, lam[i], lam[j]))
        for r in range(3):
            v[r][i], v[r][j] = (jnp.where(cond, v[r][j], v[r][i]),
                                jnp.where(cond, v[r][i], v[r][j]))

    cswap(0, 1)
    cswap(0, 2)
    cswap(1, 2)
    detv = (v[0][0] * (v[1][1] * v[2][2] - v[1][2] * v[2][1])
            - v[0][1] * (v[1][0] * v[2][2] - v[1][2] * v[2][0])
            + v[0][2] * (v[1][0] * v[2][1] - v[1][1] * v[2][0]))
    sgn = jnp.where(detv < 0, -1.0, 1.0)
    u1 = [sum(m[r][c] * v[c][0] for c in range(3)) for r in range(3)]
    u2 = [sum(m[r][c] * v[c][1] for c in range(3)) for r in range(3)]
    n1 = jax.lax.rsqrt(u1[0] * u1[0] + u1[1] * u1[1] + u1[2] * u1[2])
    n2 = jax.lax.rsqrt(u2[0] * u2[0] + u2[1] * u2[1] + u2[2] * u2[2])
    u1 = [e * n1 for e in u1]
    u2 = [e * n2 for e in u2]
    u3 = [sgn * (u1[1] * u2[2] - u1[2] * u2[1]),
          sgn * (u1[2] * u2[0] - u1[0] * u2[2]),
          sgn * (u1[0] * u2[1] - u1[1] * u2[0])]
    rows = [u1[r] * v[c][0] + u2[r] * v[c][1] + u3[r] * v[c][2]
            for r in range(3) for c in range(3)]
    rows.extend(tr)
    out_ref[...] = jnp.stack(rows, axis=0)


_TC_SUB = 32
_svd_call = pl.pallas_call(
    _svd_body,
    out_shape=jax.ShapeDtypeStruct((12, (B * N) // (_TC_SUB * 128),
                                    _TC_SUB, 128), jnp.float32),
    grid=((B * N) // (_TC_SUB * 128),),
    in_specs=[pl.BlockSpec((13, 1, _TC_SUB, 128), lambda i: (0, i, 0, 0))],
    out_specs=pl.BlockSpec((12, 1, _TC_SUB, 128), lambda i: (0, i, 0, 0)),
)


def kernel(rot, trans, pair_rot, pair_trans, confidences, topology):
    rot_f = (rot.transpose(2, 3, 0, 1).reshape(9, B, NT, 128)
             .transpose(0, 2, 1, 3))
    trans_f = (trans.transpose(2, 0, 1).reshape(3, B, NT, 128)
               .transpose(0, 2, 1, 3))
    prot_f = (pair_rot.transpose(0, 3, 4, 2, 1)
              .reshape(B, 3, 3, 4, 8, NT, 128)
              .transpose(0, 1, 2, 3, 5, 4, 6).reshape(B * 9, 4, NT, 8, 128))
    ptrans_f = (pair_trans.transpose(0, 3, 2, 1).reshape(B, 3, 4, 8, NT, 128)
                .transpose(0, 1, 2, 4, 3, 5).reshape(B * 3, 4, NT, 8, 128))
    conf_f = (confidences.transpose(0, 2, 3, 1)
              .reshape(B * K, NT, 128))
    topo_f = (topology.astype(jnp.int32).transpose(0, 2, 1)
              .reshape(B, 4, 8, NT, 128).transpose(0, 1, 3, 2, 4)
              .reshape(B * 4, NT, 8, 128))
    acc = _sc_call()(rot_f, trans_f, prot_f, ptrans_f, conf_f, topo_f)
    out = _svd_call(acc.reshape(13, (B * N) // (_TC_SUB * 128), _TC_SUB, 128))
    out = out.reshape(12, B * N)
    out_rot = out[:9].reshape(3, 3, B, N).transpose(2, 3, 0, 1)
    out_trans = out[9:12].reshape(3, B, N).transpose(1, 2, 0)
    return out_rot, out_trans
